# baseline (speedup 1.0000x reference)
import functools

import jax
import jax.numpy as jnp
from jax import lax
from jax.experimental import pallas as pl
from jax.experimental.pallas import tpu as pltpu
from jax.experimental.pallas import tpu_sc as plsc

NC = 2
NS = 16
NW = NC * NS

TOTAL = 131072
DIM = 64
CHUNK = 128
PER_W = TOTAL // NW
NCHUNK = PER_W // CHUNK

_mesh = plsc.VectorSubcoreMesh(core_axis_name="c", subcore_axis_name="s")


@functools.partial(
    pl.kernel,
    out_type=jax.ShapeDtypeStruct((TOTAL, DIM), jnp.float32),
    mesh=_mesh,
    scratch_types=dict(
        idx_v=pltpu.VMEM((NCHUNK, CHUNK), jnp.int32),
        rows_v=pltpu.VMEM((CHUNK, DIM), jnp.float32),
        gsem=pltpu.SemaphoreType.DMA,
    ),
    compiler_params=pltpu.CompilerParams(use_tc_tiling_on_sc=False),
)
def _gather_kernel(vals_hbm, table_hbm, out_hbm, idx_v, rows_v, gsem):
    wid = lax.axis_index("s") * NC + lax.axis_index("c")
    base = wid * PER_W
    pltpu.sync_copy(vals_hbm.at[wid], idx_v)

    @pl.loop(0, NCHUNK)
    def _chunk(c):
        pltpu.async_copy(table_hbm.at[idx_v.at[c]], rows_v, gsem).wait()
        pltpu.sync_copy(rows_v, out_hbm.at[pl.ds(base + c * CHUNK, CHUNK)])


def kernel(values, lengths, table):
    del lengths
    vals = values.astype(jnp.int32).reshape(NW, NCHUNK, CHUNK)
    return _gather_kernel(vals, table)


# speedup vs baseline: 1.0315x; 1.0315x over previous
import functools

import jax
import jax.numpy as jnp
from jax import lax
from jax.experimental import pallas as pl
from jax.experimental.pallas import tpu as pltpu
from jax.experimental.pallas import tpu_sc as plsc

NC = 2
NS = 16
NW = NC * NS

TOTAL = 131072
DIM = 64
CHUNK = 128
PER_W = TOTAL // NW
NCHUNK = PER_W // CHUNK
NBUF = 8
K = 4

_mesh = plsc.VectorSubcoreMesh(core_axis_name="c", subcore_axis_name="s")


@functools.partial(
    pl.kernel,
    out_type=jax.ShapeDtypeStruct((TOTAL, DIM), jnp.float32),
    mesh=_mesh,
    scratch_types=dict(
        idx_v=pltpu.VMEM((NCHUNK, CHUNK), jnp.int32),
        rows_v=pltpu.VMEM((NBUF, CHUNK, DIM), jnp.float32),
        gsems=pltpu.SemaphoreType.DMA((NBUF,)),
        ssems=pltpu.SemaphoreType.DMA((NBUF,)),
    ),
    compiler_params=pltpu.CompilerParams(use_tc_tiling_on_sc=False),
)
def _gather_kernel(vals_hbm, table_hbm, out_hbm, idx_v, rows_v, gsems, ssems):
    wid = lax.axis_index("s") * NC + lax.axis_index("c")
    base = wid * PER_W
    pltpu.sync_copy(vals_hbm.at[wid], idx_v)

    def gd(c, b):
        return pltpu.make_async_copy(table_hbm.at[idx_v.at[c]], rows_v.at[b],
                                     gsems.at[b])

    def sd(c, b):
        return pltpu.make_async_copy(rows_v.at[b],
                                     out_hbm.at[pl.ds(base + c * CHUNK, CHUNK)],
                                     ssems.at[b])

    def step(c, b, do_wait_store, do_next_gather):
        gd(c, b).wait()
        sd(c, b).start()
        if do_next_gather:
            b2 = (b + K) % NBUF
            if do_wait_store:
                sd(c - K, b2).wait()
            gd(c + K, b2).start()

    for j in range(K):
        gd(j, j).start()

    for j in range(NBUF):
        step(j, j, do_wait_store=(j >= K), do_next_gather=True)

    @pl.loop(NBUF, NCHUNK - NBUF, step=NBUF)
    def _blk(c0):
        for j in range(NBUF):
            step(c0 + j, j, do_wait_store=True, do_next_gather=True)

    for j in range(NBUF):
        c = NCHUNK - NBUF + j
        step(c, j, do_wait_store=True, do_next_gather=(j < NBUF - K))

    for j in range(NBUF):
        sd(NCHUNK - NBUF + j, j).wait()


def kernel(values, lengths, table):
    del lengths
    vals = values.astype(jnp.int32).reshape(NW, NCHUNK, CHUNK)
    return _gather_kernel(vals, table)
